# baseline (speedup 1.0000x reference)
import functools
import jax
import jax.numpy as jnp
from jax import lax
from jax.experimental import pallas as pl
from jax.experimental.pallas import tpu as pltpu
from jax.experimental.pallas import tpu_sc as plsc

N = 10000
NPAD = 10240
D = 128
NC = 2
NS = 16
NW = NC * NS
BLK = 128
NBLK = 80
EPAD = NW * NBLK * BLK
RPT = NPAD // NS
SENT = 10100
RB = 512
GRID = NPAD // RB

_mesh = plsc.VectorSubcoreMesh(core_axis_name="c", subcore_axis_name="s")

SH = 3
NBINS = 10240 >> SH
E0 = 112 * BLK
E1 = 48 * BLK


def _bin_tile(ecnt, ebase, srcf, dstf, outs, outd,
              src_in, dst_in, src_out, dst_out, hist, basep):
    pltpu.sync_copy(srcf.at[pl.ds(ebase, ecnt)], src_in.at[pl.ds(0, ecnt)])
    pltpu.sync_copy(dstf.at[pl.ds(ebase, ecnt)], dst_in.at[pl.ds(0, ecnt)])
    lanes = lax.iota(jnp.int32, 16)
    zeros16 = jnp.zeros((16,), jnp.int32)
    ones16 = jnp.ones((16,), jnp.int32)

    @pl.loop(0, NBINS)
    def _(cb):
        plsc.store_scatter(hist, [lanes, jnp.full((16,), cb, jnp.int32)],
                           zeros16)

    @pl.loop(0, ecnt // 16)
    def _(v):
        sv = src_in[pl.ds(v * 16, 16)]
        b = lax.shift_right_logical(sv, SH)
        plsc.addupdate_scatter(hist, [lanes, b], ones16)

    def pbody(cb, carry):
        bvec = jnp.full((16,), cb, jnp.int32)
        col = plsc.load_gather(hist, [lanes, bvec])
        inc = plsc.cumsum(col)
        plsc.store_scatter(basep, [lanes, bvec], carry + inc - col)
        return carry + jnp.sum(col)

    lax.fori_loop(0, NBINS, pbody, jnp.int32(0))

    @pl.loop(0, ecnt // 16)
    def _(v):
        sv = src_in[pl.ds(v * 16, 16)]
        dv = dst_in[pl.ds(v * 16, 16)]
        b = lax.shift_right_logical(sv, SH)
        pos = plsc.load_gather(basep, [lanes, b])
        plsc.store_scatter(src_out, [pos], sv)
        plsc.store_scatter(dst_out, [pos], dv)
        plsc.store_scatter(basep, [lanes, b], pos + ones16)

    pltpu.sync_copy(src_out.at[pl.ds(0, ecnt)], outs.at[pl.ds(ebase, ecnt)])
    pltpu.sync_copy(dst_out.at[pl.ds(0, ecnt)], outd.at[pl.ds(ebase, ecnt)])


def _bin_body(srcf, dstf, outs, outd,
              src_in, dst_in, src_out, dst_out, hist, basep):
    c = lax.axis_index("c")
    s = lax.axis_index("s")

    @pl.when(c == 0)
    def _():
        _bin_tile(E0, s * E0, srcf, dstf, outs, outd,
                  src_in, dst_in, src_out, dst_out, hist, basep)

    @pl.when(c == 1)
    def _():
        _bin_tile(E1, NS * E0 + s * E1, srcf, dstf, outs, outd,
                  src_in, dst_in, src_out, dst_out, hist, basep)


_bin_call = pl.kernel(
    _bin_body,
    compiler_params=pltpu.CompilerParams(needs_layout_passes=False),
    out_type=[
        jax.ShapeDtypeStruct((NW * NBLK * BLK,), jnp.int32),
        jax.ShapeDtypeStruct((NW * NBLK * BLK,), jnp.int32),
    ],
    mesh=_mesh,
    scratch_types=[
        pltpu.VMEM((E0,), jnp.int32),
        pltpu.VMEM((E0,), jnp.int32),
        pltpu.VMEM((E0,), jnp.int32),
        pltpu.VMEM((E0,), jnp.int32),
        pltpu.VMEM((16, NBINS), jnp.int32),
        pltpu.VMEM((16, NBINS), jnp.int32),
    ],
)


def _deg_body(dstb_hbm, zeros1_hbm, out_hbm, dst_v, ones_v, hist_sh, sem):
    c = lax.axis_index("c")
    s = lax.axis_index("s")
    wid = c * NS + s
    pltpu.sync_copy(dstb_hbm.at[pl.ds(wid * NBLK, NBLK)], dst_v)
    for k in range(BLK // 16):
        ones_v[pl.ds(k * 16, 16)] = jnp.ones((16,), jnp.float32)

    @pl.when(s == 0)
    def _():
        pltpu.sync_copy(zeros1_hbm, hist_sh)

    plsc.subcore_barrier()

    def body(j, carry):
        pltpu.async_copy(ones_v, hist_sh.at[dst_v.at[j]], sem, add=True).wait()
        return carry

    lax.fori_loop(0, NBLK, body, 0)
    plsc.subcore_barrier()

    @pl.when(s == 0)
    def _():
        pltpu.sync_copy(hist_sh, out_hbm.at[c])


_deg_call = pl.kernel(
    _deg_body,
    out_type=jax.ShapeDtypeStruct((NC, NPAD), jnp.float32),
    mesh=_mesh,
    scratch_types=[
        pltpu.VMEM((NBLK, BLK), jnp.int32),
        pltpu.VMEM((BLK,), jnp.float32),
        pltpu.VMEM_SHARED((NPAD,), jnp.float32),
        pltpu.SemaphoreType.DMA,
    ],
)


NBUF = 2
CHUNK = 16
B0 = 112
B1 = 48
TOTBLK = NS * (B0 + B1)


def _msg_ring(nb, base, g_hbm, srcb_hbm, dstb_hbm,
              src_v, dst_v, rows_v, acc_sh, gsems, ssems):
    def start_gather(j, b):
        pltpu.async_copy(g_hbm.at[src_v.at[j]], rows_v.at[b], gsems[b])

    def wait_gather(j, b):
        pltpu.make_async_copy(g_hbm.at[src_v.at[j]], rows_v.at[b],
                              gsems[b]).wait()

    def start_scatter(j, b):
        pltpu.async_copy(rows_v.at[b], acc_sh.at[dst_v.at[j]], ssems[b],
                         add=True)

    def wait_scatter(j, b):
        pltpu.make_async_copy(rows_v.at[b], acc_sh.at[dst_v.at[j]],
                              ssems[b]).wait()

    for h in range(nb // CHUNK):
        pltpu.sync_copy(srcb_hbm.at[pl.ds(base + h * CHUNK, CHUNK)], src_v)
        pltpu.sync_copy(dstb_hbm.at[pl.ds(base + h * CHUNK, CHUNK)], dst_v)

        for b in range(NBUF - 1):
            start_gather(b, b)
        for j in range(NBUF):
            wait_gather(j, j)
            start_scatter(j, j)
            if j > 0:
                wait_scatter(j - 1, (j - 1) % NBUF)
            start_gather(j + NBUF - 1, (j + NBUF - 1) % NBUF)

        @pl.loop(NBUF, CHUNK, step=NBUF)
        def _(j0):
            for b in range(NBUF):
                j = j0 + b
                wait_gather(j, b)
                start_scatter(j, b)
                nb2 = (b + NBUF - 1) % NBUF

                @pl.when(j + NBUF - 1 < CHUNK)
                def _():
                    wait_scatter(j - 1, nb2)
                    start_gather(j + NBUF - 1, nb2)

        for b in range(NBUF):
            j = CHUNK - NBUF + b
            wait_scatter(j, j % NBUF)


def _msg_body(g_hbm, srcb_hbm, dstb_hbm, zeros2_hbm, out_hbm,
              src_v, dst_v, rows_v, acc_sh, *sems):
    gsems = sems[:NBUF]
    ssems = sems[NBUF:]
    c = lax.axis_index("c")
    s = lax.axis_index("s")
    pltpu.sync_copy(zeros2_hbm.at[pl.ds(s * RPT, RPT)],
                    acc_sh.at[pl.ds(s * RPT, RPT)])
    plsc.subcore_barrier()

    @pl.when(c == 0)
    def _():
        _msg_ring(B0, s * B0, g_hbm, srcb_hbm, dstb_hbm,
                  src_v, dst_v, rows_v, acc_sh, gsems, ssems)

    @pl.when(c == 1)
    def _():
        _msg_ring(B1, NS * B0 + s * B1, g_hbm, srcb_hbm, dstb_hbm,
                  src_v, dst_v, rows_v, acc_sh, gsems, ssems)

    plsc.subcore_barrier()
    pltpu.sync_copy(acc_sh.at[pl.ds(s * RPT, RPT)],
                    out_hbm.at[c, pl.ds(s * RPT, RPT)])


_msg_call = pl.kernel(
    _msg_body,
    out_type=jax.ShapeDtypeStruct((NC, NPAD, D), jnp.float32),
    mesh=_mesh,
    scratch_types=[
        pltpu.VMEM((CHUNK, BLK), jnp.int32),
        pltpu.VMEM((CHUNK, BLK), jnp.int32),
        pltpu.VMEM((NBUF, BLK, D), jnp.float32),
        pltpu.VMEM_SHARED((NPAD, D), jnp.float32),
    ] + [pltpu.SemaphoreType.DMA] * (2 * NBUF),
)


def _m1_body(x_ref, w_ref, hist_ref, g_ref, dinv_ref):
    i = pl.program_id(0)
    h = jnp.dot(x_ref[...], w_ref[...], preferred_element_type=jnp.float32)
    deg = hist_ref[...].sum(axis=1, keepdims=True) + 1.0
    rid = lax.broadcasted_iota(jnp.int32, (RB, 1), 0) + i * RB
    dinv = jnp.where(rid < N, lax.rsqrt(deg), 0.0)
    g_ref[...] = dinv * h
    dinv_ref[...] = dinv


_m1_call = pl.pallas_call(
    _m1_body,
    grid=(GRID,),
    in_specs=[
        pl.BlockSpec((RB, D), lambda i: (i, 0)),
        pl.BlockSpec((D, D), lambda i: (0, 0)),
        pl.BlockSpec((RB, NC), lambda i: (i, 0)),
    ],
    out_specs=[
        pl.BlockSpec((RB, D), lambda i: (i, 0)),
        pl.BlockSpec((RB, 1), lambda i: (i, 0)),
    ],
    out_shape=[
        jax.ShapeDtypeStruct((NPAD, D), jnp.float32),
        jax.ShapeDtypeStruct((NPAD, 1), jnp.float32),
    ],
)


def _m2_body(s_ref, g_ref, dinv_ref, b_ref, w_ref, out_ref):
    dinv = dinv_ref[...]
    z = jnp.maximum(dinv * (s_ref[0] + s_ref[1] + g_ref[...]) + b_ref[...],
                    0.0)
    out_ref[...] = dinv * jnp.dot(z, w_ref[...],
                                  preferred_element_type=jnp.float32)


_m2_call = pl.pallas_call(
    _m2_body,
    grid=(GRID,),
    in_specs=[
        pl.BlockSpec((NC, RB, D), lambda i: (0, i, 0)),
        pl.BlockSpec((RB, D), lambda i: (i, 0)),
        pl.BlockSpec((RB, 1), lambda i: (i, 0)),
        pl.BlockSpec((1, D), lambda i: (0, 0)),
        pl.BlockSpec((D, D), lambda i: (0, 0)),
    ],
    out_specs=pl.BlockSpec((RB, D), lambda i: (i, 0)),
    out_shape=jax.ShapeDtypeStruct((NPAD, D), jnp.float32),
)


def _m3_body(s_ref, g_ref, dinv_ref, b_ref, w_ref, bp_ref, out_ref):
    h = (dinv_ref[...] * (s_ref[0] + s_ref[1] + g_ref[...]) + b_ref[...])
    out_ref[...] = jnp.dot(h, w_ref[...],
                           preferred_element_type=jnp.float32) + bp_ref[...]


_m3_call = pl.pallas_call(
    _m3_body,
    grid=(GRID,),
    in_specs=[
        pl.BlockSpec((NC, RB, D), lambda i: (0, i, 0)),
        pl.BlockSpec((RB, D), lambda i: (i, 0)),
        pl.BlockSpec((RB, 1), lambda i: (i, 0)),
        pl.BlockSpec((1, D), lambda i: (0, 0)),
        pl.BlockSpec((D, D), lambda i: (0, 0)),
        pl.BlockSpec((1, D), lambda i: (0, 0)),
    ],
    out_specs=pl.BlockSpec((RB, D), lambda i: (i, 0)),
    out_shape=jax.ShapeDtypeStruct((NPAD, D), jnp.float32),
)


@jax.jit
def kernel(x, edge_index, W1, b1, W2, b2, Wp, bp):
    src = edge_index[0].astype(jnp.int32)
    dst = edge_index[1].astype(jnp.int32)
    e = src.shape[0]
    fill = jnp.full((EPAD - e,), SENT, jnp.int32)
    src_f = jnp.concatenate([src, fill])
    dst_f = jnp.concatenate([dst, fill])
    bsrc, bdst = _bin_call(src_f, dst_f)
    src_p = bsrc.reshape(TOTBLK, BLK)
    dst_p = bdst.reshape(TOTBLK, BLK)
    x_p = jnp.pad(x, ((0, NPAD - N), (0, 0)))
    zeros1 = jnp.zeros((NPAD,), jnp.float32)
    zeros2 = jnp.zeros((NPAD, D), jnp.float32)

    hist = _deg_call(dst_p, zeros1)
    g1, dinv = _m1_call(x_p, W1, hist.T)
    s1 = _msg_call(g1, src_p, dst_p, zeros2)
    g2 = _m2_call(s1, g1, dinv, b1.reshape(1, D), W2)
    s2 = _msg_call(g2, src_p, dst_p, zeros2)
    out = _m3_call(s2, g2, dinv, b2.reshape(1, D), Wp, bp.reshape(1, D))
    return out[:N]


# speedup vs baseline: 1.0473x; 1.0473x over previous
import functools
import jax
import jax.numpy as jnp
from jax import lax
from jax.experimental import pallas as pl
from jax.experimental.pallas import tpu as pltpu
from jax.experimental.pallas import tpu_sc as plsc

N = 10000
NPAD = 10240
D = 128
NC = 2
NS = 16
NW = NC * NS
BLK = 128
NBLK = 80
EPAD = NW * NBLK * BLK
RPT = NPAD // NS
SENT = 10100
RB = 512
GRID = NPAD // RB

_mesh = plsc.VectorSubcoreMesh(core_axis_name="c", subcore_axis_name="s")

SH = 4
NBINS = 10240 >> SH
E0 = 128 * BLK
E1 = 32 * BLK


def _bin_tile(ecnt, ebase, srcf, dstf, outs, outd,
              src_in, dst_in, src_out, dst_out, hist, basep):
    pltpu.sync_copy(srcf.at[pl.ds(ebase, ecnt)], src_in.at[pl.ds(0, ecnt)])
    pltpu.sync_copy(dstf.at[pl.ds(ebase, ecnt)], dst_in.at[pl.ds(0, ecnt)])
    lanes = lax.iota(jnp.int32, 16)
    zeros16 = jnp.zeros((16,), jnp.int32)
    ones16 = jnp.ones((16,), jnp.int32)

    @pl.loop(0, NBINS)
    def _(cb):
        plsc.store_scatter(hist, [lanes, jnp.full((16,), cb, jnp.int32)],
                           zeros16)

    @pl.loop(0, ecnt // 16)
    def _(v):
        sv = src_in[pl.ds(v * 16, 16)]
        b = lax.shift_right_logical(sv, SH)
        plsc.addupdate_scatter(hist, [lanes, b], ones16)

    def pbody(cb, carry):
        bvec = jnp.full((16,), cb, jnp.int32)
        col = plsc.load_gather(hist, [lanes, bvec])
        inc = plsc.cumsum(col)
        plsc.store_scatter(basep, [lanes, bvec], carry + inc - col)
        return carry + jnp.sum(col)

    lax.fori_loop(0, NBINS, pbody, jnp.int32(0))

    @pl.loop(0, ecnt // 16)
    def _(v):
        sv = src_in[pl.ds(v * 16, 16)]
        dv = dst_in[pl.ds(v * 16, 16)]
        b = lax.shift_right_logical(sv, SH)
        pos = plsc.load_gather(basep, [lanes, b])
        plsc.store_scatter(src_out, [pos], sv)
        plsc.store_scatter(dst_out, [pos], dv)
        plsc.store_scatter(basep, [lanes, b], pos + ones16)

    pltpu.sync_copy(src_out.at[pl.ds(0, ecnt)], outs.at[pl.ds(ebase, ecnt)])
    pltpu.sync_copy(dst_out.at[pl.ds(0, ecnt)], outd.at[pl.ds(ebase, ecnt)])


def _bin_body(srcf, dstf, outs, outd,
              src_in, dst_in, src_out, dst_out, hist, basep):
    c = lax.axis_index("c")
    s = lax.axis_index("s")

    @pl.when(c == 0)
    def _():
        _bin_tile(E0, s * E0, srcf, dstf, outs, outd,
                  src_in, dst_in, src_out, dst_out, hist, basep)

    @pl.when(c == 1)
    def _():
        _bin_tile(E1, NS * E0 + s * E1, srcf, dstf, outs, outd,
                  src_in, dst_in, src_out, dst_out, hist, basep)


_bin_call = pl.kernel(
    _bin_body,
    compiler_params=pltpu.CompilerParams(needs_layout_passes=False),
    out_type=[
        jax.ShapeDtypeStruct((NW * NBLK * BLK,), jnp.int32),
        jax.ShapeDtypeStruct((NW * NBLK * BLK,), jnp.int32),
    ],
    mesh=_mesh,
    scratch_types=[
        pltpu.VMEM((E0,), jnp.int32),
        pltpu.VMEM((E0,), jnp.int32),
        pltpu.VMEM((E0,), jnp.int32),
        pltpu.VMEM((E0,), jnp.int32),
        pltpu.VMEM((16, NBINS), jnp.int32),
        pltpu.VMEM((16, NBINS), jnp.int32),
    ],
)


def _deg_body(dstb_hbm, zeros1_hbm, out_hbm, dst_v, ones_v, hist_sh, sem):
    c = lax.axis_index("c")
    s = lax.axis_index("s")
    wid = c * NS + s
    pltpu.sync_copy(dstb_hbm.at[pl.ds(wid * NBLK, NBLK)], dst_v)
    for k in range(BLK // 16):
        ones_v[pl.ds(k * 16, 16)] = jnp.ones((16,), jnp.float32)

    @pl.when(s == 0)
    def _():
        pltpu.sync_copy(zeros1_hbm, hist_sh)

    plsc.subcore_barrier()

    def body(j, carry):
        pltpu.async_copy(ones_v, hist_sh.at[dst_v.at[j]], sem, add=True).wait()
        return carry

    lax.fori_loop(0, NBLK, body, 0)
    plsc.subcore_barrier()

    @pl.when(s == 0)
    def _():
        pltpu.sync_copy(hist_sh, out_hbm.at[c])


_deg_call = pl.kernel(
    _deg_body,
    out_type=jax.ShapeDtypeStruct((NC, NPAD), jnp.float32),
    mesh=_mesh,
    scratch_types=[
        pltpu.VMEM((NBLK, BLK), jnp.int32),
        pltpu.VMEM((BLK,), jnp.float32),
        pltpu.VMEM_SHARED((NPAD,), jnp.float32),
        pltpu.SemaphoreType.DMA,
    ],
)


NBUF = 2
CHUNK = 16
B0 = 128
B1 = 32
TOTBLK = NS * (B0 + B1)


def _msg_ring(nb, base, g_hbm, srcb_hbm, dstb_hbm,
              src_v, dst_v, rows_v, acc_sh, gsems, ssems):
    def start_gather(j, b):
        pltpu.async_copy(g_hbm.at[src_v.at[j]], rows_v.at[b], gsems[b])

    def wait_gather(j, b):
        pltpu.make_async_copy(g_hbm.at[src_v.at[j]], rows_v.at[b],
                              gsems[b]).wait()

    def start_scatter(j, b):
        pltpu.async_copy(rows_v.at[b], acc_sh.at[dst_v.at[j]], ssems[b],
                         add=True)

    def wait_scatter(j, b):
        pltpu.make_async_copy(rows_v.at[b], acc_sh.at[dst_v.at[j]],
                              ssems[b]).wait()

    for h in range(nb // CHUNK):
        pltpu.sync_copy(srcb_hbm.at[pl.ds(base + h * CHUNK, CHUNK)], src_v)
        pltpu.sync_copy(dstb_hbm.at[pl.ds(base + h * CHUNK, CHUNK)], dst_v)

        for b in range(NBUF - 1):
            start_gather(b, b)
        for j in range(NBUF):
            wait_gather(j, j)
            start_scatter(j, j)
            if j > 0:
                wait_scatter(j - 1, (j - 1) % NBUF)
            start_gather(j + NBUF - 1, (j + NBUF - 1) % NBUF)

        @pl.loop(NBUF, CHUNK, step=NBUF)
        def _(j0):
            for b in range(NBUF):
                j = j0 + b
                wait_gather(j, b)
                start_scatter(j, b)
                nb2 = (b + NBUF - 1) % NBUF

                @pl.when(j + NBUF - 1 < CHUNK)
                def _():
                    wait_scatter(j - 1, nb2)
                    start_gather(j + NBUF - 1, nb2)

        for b in range(NBUF):
            j = CHUNK - NBUF + b
            wait_scatter(j, j % NBUF)


def _msg_body(g_hbm, srcb_hbm, dstb_hbm, zeros2_hbm, out_hbm,
              src_v, dst_v, rows_v, acc_sh, *sems):
    gsems = sems[:NBUF]
    ssems = sems[NBUF:]
    c = lax.axis_index("c")
    s = lax.axis_index("s")
    pltpu.sync_copy(zeros2_hbm.at[pl.ds(s * RPT, RPT)],
                    acc_sh.at[pl.ds(s * RPT, RPT)])
    plsc.subcore_barrier()

    @pl.when(c == 0)
    def _():
        _msg_ring(B0, s * B0, g_hbm, srcb_hbm, dstb_hbm,
                  src_v, dst_v, rows_v, acc_sh, gsems, ssems)

    @pl.when(c == 1)
    def _():
        _msg_ring(B1, NS * B0 + s * B1, g_hbm, srcb_hbm, dstb_hbm,
                  src_v, dst_v, rows_v, acc_sh, gsems, ssems)

    plsc.subcore_barrier()
    pltpu.sync_copy(acc_sh.at[pl.ds(s * RPT, RPT)],
                    out_hbm.at[c, pl.ds(s * RPT, RPT)])


_msg_call = pl.kernel(
    _msg_body,
    out_type=jax.ShapeDtypeStruct((NC, NPAD, D), jnp.float32),
    mesh=_mesh,
    scratch_types=[
        pltpu.VMEM((CHUNK, BLK), jnp.int32),
        pltpu.VMEM((CHUNK, BLK), jnp.int32),
        pltpu.VMEM((NBUF, BLK, D), jnp.float32),
        pltpu.VMEM_SHARED((NPAD, D), jnp.float32),
    ] + [pltpu.SemaphoreType.DMA] * (2 * NBUF),
)


def _m1_body(x_ref, w_ref, hist_ref, g_ref, dinv_ref):
    i = pl.program_id(0)
    h = jnp.dot(x_ref[...], w_ref[...], preferred_element_type=jnp.float32)
    deg = hist_ref[...].sum(axis=1, keepdims=True) + 1.0
    rid = lax.broadcasted_iota(jnp.int32, (RB, 1), 0) + i * RB
    dinv = jnp.where(rid < N, lax.rsqrt(deg), 0.0)
    g_ref[...] = dinv * h
    dinv_ref[...] = dinv


_m1_call = pl.pallas_call(
    _m1_body,
    grid=(GRID,),
    in_specs=[
        pl.BlockSpec((RB, D), lambda i: (i, 0)),
        pl.BlockSpec((D, D), lambda i: (0, 0)),
        pl.BlockSpec((RB, NC), lambda i: (i, 0)),
    ],
    out_specs=[
        pl.BlockSpec((RB, D), lambda i: (i, 0)),
        pl.BlockSpec((RB, 1), lambda i: (i, 0)),
    ],
    out_shape=[
        jax.ShapeDtypeStruct((NPAD, D), jnp.float32),
        jax.ShapeDtypeStruct((NPAD, 1), jnp.float32),
    ],
)


def _m2_body(s_ref, g_ref, dinv_ref, b_ref, w_ref, out_ref):
    dinv = dinv_ref[...]
    z = jnp.maximum(dinv * (s_ref[0] + s_ref[1] + g_ref[...]) + b_ref[...],
                    0.0)
    out_ref[...] = dinv * jnp.dot(z, w_ref[...],
                                  preferred_element_type=jnp.float32)


_m2_call = pl.pallas_call(
    _m2_body,
    grid=(GRID,),
    in_specs=[
        pl.BlockSpec((NC, RB, D), lambda i: (0, i, 0)),
        pl.BlockSpec((RB, D), lambda i: (i, 0)),
        pl.BlockSpec((RB, 1), lambda i: (i, 0)),
        pl.BlockSpec((1, D), lambda i: (0, 0)),
        pl.BlockSpec((D, D), lambda i: (0, 0)),
    ],
    out_specs=pl.BlockSpec((RB, D), lambda i: (i, 0)),
    out_shape=jax.ShapeDtypeStruct((NPAD, D), jnp.float32),
)


def _m3_body(s_ref, g_ref, dinv_ref, b_ref, w_ref, bp_ref, out_ref):
    h = (dinv_ref[...] * (s_ref[0] + s_ref[1] + g_ref[...]) + b_ref[...])
    out_ref[...] = jnp.dot(h, w_ref[...],
                           preferred_element_type=jnp.float32) + bp_ref[...]


_m3_call = pl.pallas_call(
    _m3_body,
    grid=(GRID,),
    in_specs=[
        pl.BlockSpec((NC, RB, D), lambda i: (0, i, 0)),
        pl.BlockSpec((RB, D), lambda i: (i, 0)),
        pl.BlockSpec((RB, 1), lambda i: (i, 0)),
        pl.BlockSpec((1, D), lambda i: (0, 0)),
        pl.BlockSpec((D, D), lambda i: (0, 0)),
        pl.BlockSpec((1, D), lambda i: (0, 0)),
    ],
    out_specs=pl.BlockSpec((RB, D), lambda i: (i, 0)),
    out_shape=jax.ShapeDtypeStruct((NPAD, D), jnp.float32),
)


@jax.jit
def kernel(x, edge_index, W1, b1, W2, b2, Wp, bp):
    src = edge_index[0].astype(jnp.int32)
    dst = edge_index[1].astype(jnp.int32)
    e = src.shape[0]
    fill = jnp.full((EPAD - e,), SENT, jnp.int32)
    src_f = jnp.concatenate([src, fill])
    dst_f = jnp.concatenate([dst, fill])
    bsrc, bdst = _bin_call(src_f, dst_f)
    src_p = bsrc.reshape(TOTBLK, BLK)
    dst_p = bdst.reshape(TOTBLK, BLK)
    x_p = jnp.pad(x, ((0, NPAD - N), (0, 0)))
    zeros1 = jnp.zeros((NPAD,), jnp.float32)
    zeros2 = jnp.zeros((NPAD, D), jnp.float32)

    hist = _deg_call(dst_p, zeros1)
    g1, dinv = _m1_call(x_p, W1, hist.T)
    s1 = _msg_call(g1, src_p, dst_p, zeros2)
    g2 = _m2_call(s1, g1, dinv, b1.reshape(1, D), W2)
    s2 = _msg_call(g2, src_p, dst_p, zeros2)
    out = _m3_call(s2, g2, dinv, b2.reshape(1, D), Wp, bp.reshape(1, D))
    return out[:N]


# speedup vs baseline: 1.0570x; 1.0092x over previous
import functools
import jax
import jax.numpy as jnp
from jax import lax
from jax.experimental import pallas as pl
from jax.experimental.pallas import tpu as pltpu
from jax.experimental.pallas import tpu_sc as plsc

N = 10000
NPAD = 10240
D = 128
NC = 2
NS = 16
NW = NC * NS
BLK = 128
NBLK = 80
EPAD = NW * NBLK * BLK
RPT = NPAD // NS
SENT = 10100
RB = 512
GRID = NPAD // RB

_mesh = plsc.VectorSubcoreMesh(core_axis_name="c", subcore_axis_name="s")

SH = 5
NBINS = 10240 >> SH
E0 = 128 * BLK
E1 = 32 * BLK


def _bin_tile(ecnt, ebase, srcf, dstf, outs, outd,
              src_in, dst_in, src_out, dst_out, hist, basep):
    pltpu.sync_copy(srcf.at[pl.ds(ebase, ecnt)], src_in.at[pl.ds(0, ecnt)])
    pltpu.sync_copy(dstf.at[pl.ds(ebase, ecnt)], dst_in.at[pl.ds(0, ecnt)])
    lanes = lax.iota(jnp.int32, 16)
    zeros16 = jnp.zeros((16,), jnp.int32)
    ones16 = jnp.ones((16,), jnp.int32)

    @pl.loop(0, NBINS)
    def _(cb):
        plsc.store_scatter(hist, [lanes, jnp.full((16,), cb, jnp.int32)],
                           zeros16)

    @pl.loop(0, ecnt // 16)
    def _(v):
        sv = src_in[pl.ds(v * 16, 16)]
        b = lax.shift_right_logical(sv, SH)
        plsc.addupdate_scatter(hist, [lanes, b], ones16)

    def pbody(cb, carry):
        bvec = jnp.full((16,), cb, jnp.int32)
        col = plsc.load_gather(hist, [lanes, bvec])
        inc = plsc.cumsum(col)
        plsc.store_scatter(basep, [lanes, bvec], carry + inc - col)
        return carry + jnp.sum(col)

    lax.fori_loop(0, NBINS, pbody, jnp.int32(0))

    @pl.loop(0, ecnt // 16)
    def _(v):
        sv = src_in[pl.ds(v * 16, 16)]
        dv = dst_in[pl.ds(v * 16, 16)]
        b = lax.shift_right_logical(sv, SH)
        pos = plsc.load_gather(basep, [lanes, b])
        plsc.store_scatter(src_out, [pos], sv)
        plsc.store_scatter(dst_out, [pos], dv)
        plsc.store_scatter(basep, [lanes, b], pos + ones16)

    pltpu.sync_copy(src_out.at[pl.ds(0, ecnt)], outs.at[pl.ds(ebase, ecnt)])
    pltpu.sync_copy(dst_out.at[pl.ds(0, ecnt)], outd.at[pl.ds(ebase, ecnt)])


def _bin_body(srcf, dstf, outs, outd,
              src_in, dst_in, src_out, dst_out, hist, basep):
    c = lax.axis_index("c")
    s = lax.axis_index("s")

    @pl.when(c == 0)
    def _():
        _bin_tile(E0, s * E0, srcf, dstf, outs, outd,
                  src_in, dst_in, src_out, dst_out, hist, basep)

    @pl.when(c == 1)
    def _():
        _bin_tile(E1, NS * E0 + s * E1, srcf, dstf, outs, outd,
                  src_in, dst_in, src_out, dst_out, hist, basep)


_bin_call = pl.kernel(
    _bin_body,
    compiler_params=pltpu.CompilerParams(needs_layout_passes=False),
    out_type=[
        jax.ShapeDtypeStruct((NW * NBLK * BLK,), jnp.int32),
        jax.ShapeDtypeStruct((NW * NBLK * BLK,), jnp.int32),
    ],
    mesh=_mesh,
    scratch_types=[
        pltpu.VMEM((E0,), jnp.int32),
        pltpu.VMEM((E0,), jnp.int32),
        pltpu.VMEM((E0,), jnp.int32),
        pltpu.VMEM((E0,), jnp.int32),
        pltpu.VMEM((16, NBINS), jnp.int32),
        pltpu.VMEM((16, NBINS), jnp.int32),
    ],
)


def _deg_body(dstb_hbm, zeros1_hbm, out_hbm, dst_v, ones_v, hist_sh, sem):
    c = lax.axis_index("c")
    s = lax.axis_index("s")
    wid = c * NS + s
    pltpu.sync_copy(dstb_hbm.at[pl.ds(wid * NBLK, NBLK)], dst_v)
    for k in range(BLK // 16):
        ones_v[pl.ds(k * 16, 16)] = jnp.ones((16,), jnp.float32)

    @pl.when(s == 0)
    def _():
        pltpu.sync_copy(zeros1_hbm, hist_sh)

    plsc.subcore_barrier()

    def body(j, carry):
        pltpu.async_copy(ones_v, hist_sh.at[dst_v.at[j]], sem, add=True).wait()
        return carry

    lax.fori_loop(0, NBLK, body, 0)
    plsc.subcore_barrier()

    @pl.when(s == 0)
    def _():
        pltpu.sync_copy(hist_sh, out_hbm.at[c])


_deg_call = pl.kernel(
    _deg_body,
    out_type=jax.ShapeDtypeStruct((NC, NPAD), jnp.float32),
    mesh=_mesh,
    scratch_types=[
        pltpu.VMEM((NBLK, BLK), jnp.int32),
        pltpu.VMEM((BLK,), jnp.float32),
        pltpu.VMEM_SHARED((NPAD,), jnp.float32),
        pltpu.SemaphoreType.DMA,
    ],
)


NBUF = 2
CHUNK = 16
B0 = 128
B1 = 32
TOTBLK = NS * (B0 + B1)


def _msg_ring(nb, base, g_hbm, srcb_hbm, dstb_hbm,
              src_v, dst_v, rows_v, acc_sh, gsems, ssems):
    def start_gather(j, b):
        pltpu.async_copy(g_hbm.at[src_v.at[j]], rows_v.at[b], gsems[b])

    def wait_gather(j, b):
        pltpu.make_async_copy(g_hbm.at[src_v.at[j]], rows_v.at[b],
                              gsems[b]).wait()

    def start_scatter(j, b):
        pltpu.async_copy(rows_v.at[b], acc_sh.at[dst_v.at[j]], ssems[b],
                         add=True)

    def wait_scatter(j, b):
        pltpu.make_async_copy(rows_v.at[b], acc_sh.at[dst_v.at[j]],
                              ssems[b]).wait()

    for h in range(nb // CHUNK):
        pltpu.sync_copy(srcb_hbm.at[pl.ds(base + h * CHUNK, CHUNK)], src_v)
        pltpu.sync_copy(dstb_hbm.at[pl.ds(base + h * CHUNK, CHUNK)], dst_v)

        for b in range(NBUF - 1):
            start_gather(b, b)
        for j in range(NBUF):
            wait_gather(j, j)
            start_scatter(j, j)
            if j > 0:
                wait_scatter(j - 1, (j - 1) % NBUF)
            start_gather(j + NBUF - 1, (j + NBUF - 1) % NBUF)

        @pl.loop(NBUF, CHUNK, step=NBUF)
        def _(j0):
            for b in range(NBUF):
                j = j0 + b
                wait_gather(j, b)
                start_scatter(j, b)
                nb2 = (b + NBUF - 1) % NBUF

                @pl.when(j + NBUF - 1 < CHUNK)
                def _():
                    wait_scatter(j - 1, nb2)
                    start_gather(j + NBUF - 1, nb2)

        for b in range(NBUF):
            j = CHUNK - NBUF + b
            wait_scatter(j, j % NBUF)


def _msg_body(g_hbm, srcb_hbm, dstb_hbm, zeros2_hbm, out_hbm,
              src_v, dst_v, rows_v, acc_sh, *sems):
    gsems = sems[:NBUF]
    ssems = sems[NBUF:]
    c = lax.axis_index("c")
    s = lax.axis_index("s")
    pltpu.sync_copy(zeros2_hbm.at[pl.ds(s * RPT, RPT)],
                    acc_sh.at[pl.ds(s * RPT, RPT)])
    plsc.subcore_barrier()

    @pl.when(c == 0)
    def _():
        _msg_ring(B0, s * B0, g_hbm, srcb_hbm, dstb_hbm,
                  src_v, dst_v, rows_v, acc_sh, gsems, ssems)

    @pl.when(c == 1)
    def _():
        _msg_ring(B1, NS * B0 + s * B1, g_hbm, srcb_hbm, dstb_hbm,
                  src_v, dst_v, rows_v, acc_sh, gsems, ssems)

    plsc.subcore_barrier()
    pltpu.sync_copy(acc_sh.at[pl.ds(s * RPT, RPT)],
                    out_hbm.at[c, pl.ds(s * RPT, RPT)])


_msg_call = pl.kernel(
    _msg_body,
    out_type=jax.ShapeDtypeStruct((NC, NPAD, D), jnp.float32),
    mesh=_mesh,
    scratch_types=[
        pltpu.VMEM((CHUNK, BLK), jnp.int32),
        pltpu.VMEM((CHUNK, BLK), jnp.int32),
        pltpu.VMEM((NBUF, BLK, D), jnp.float32),
        pltpu.VMEM_SHARED((NPAD, D), jnp.float32),
    ] + [pltpu.SemaphoreType.DMA] * (2 * NBUF),
)


def _m1_body(x_ref, w_ref, hist_ref, g_ref, dinv_ref):
    i = pl.program_id(0)
    h = jnp.dot(x_ref[...], w_ref[...], preferred_element_type=jnp.float32)
    deg = hist_ref[...].sum(axis=1, keepdims=True) + 1.0
    rid = lax.broadcasted_iota(jnp.int32, (RB, 1), 0) + i * RB
    dinv = jnp.where(rid < N, lax.rsqrt(deg), 0.0)
    g_ref[...] = dinv * h
    dinv_ref[...] = dinv


_m1_call = pl.pallas_call(
    _m1_body,
    grid=(GRID,),
    in_specs=[
        pl.BlockSpec((RB, D), lambda i: (i, 0)),
        pl.BlockSpec((D, D), lambda i: (0, 0)),
        pl.BlockSpec((RB, NC), lambda i: (i, 0)),
    ],
    out_specs=[
        pl.BlockSpec((RB, D), lambda i: (i, 0)),
        pl.BlockSpec((RB, 1), lambda i: (i, 0)),
    ],
    out_shape=[
        jax.ShapeDtypeStruct((NPAD, D), jnp.float32),
        jax.ShapeDtypeStruct((NPAD, 1), jnp.float32),
    ],
)


def _m2_body(s_ref, g_ref, dinv_ref, b_ref, w_ref, out_ref):
    dinv = dinv_ref[...]
    z = jnp.maximum(dinv * (s_ref[0] + s_ref[1] + g_ref[...]) + b_ref[...],
                    0.0)
    out_ref[...] = dinv * jnp.dot(z, w_ref[...],
                                  preferred_element_type=jnp.float32)


_m2_call = pl.pallas_call(
    _m2_body,
    grid=(GRID,),
    in_specs=[
        pl.BlockSpec((NC, RB, D), lambda i: (0, i, 0)),
        pl.BlockSpec((RB, D), lambda i: (i, 0)),
        pl.BlockSpec((RB, 1), lambda i: (i, 0)),
        pl.BlockSpec((1, D), lambda i: (0, 0)),
        pl.BlockSpec((D, D), lambda i: (0, 0)),
    ],
    out_specs=pl.BlockSpec((RB, D), lambda i: (i, 0)),
    out_shape=jax.ShapeDtypeStruct((NPAD, D), jnp.float32),
)


def _m3_body(s_ref, g_ref, dinv_ref, b_ref, w_ref, bp_ref, out_ref):
    h = (dinv_ref[...] * (s_ref[0] + s_ref[1] + g_ref[...]) + b_ref[...])
    out_ref[...] = jnp.dot(h, w_ref[...],
                           preferred_element_type=jnp.float32) + bp_ref[...]


_m3_call = pl.pallas_call(
    _m3_body,
    grid=(GRID,),
    in_specs=[
        pl.BlockSpec((NC, RB, D), lambda i: (0, i, 0)),
        pl.BlockSpec((RB, D), lambda i: (i, 0)),
        pl.BlockSpec((RB, 1), lambda i: (i, 0)),
        pl.BlockSpec((1, D), lambda i: (0, 0)),
        pl.BlockSpec((D, D), lambda i: (0, 0)),
        pl.BlockSpec((1, D), lambda i: (0, 0)),
    ],
    out_specs=pl.BlockSpec((RB, D), lambda i: (i, 0)),
    out_shape=jax.ShapeDtypeStruct((NPAD, D), jnp.float32),
)


@jax.jit
def kernel(x, edge_index, W1, b1, W2, b2, Wp, bp):
    src = edge_index[0].astype(jnp.int32)
    dst = edge_index[1].astype(jnp.int32)
    e = src.shape[0]
    fill = jnp.full((EPAD - e,), SENT, jnp.int32)
    src_f = jnp.concatenate([src, fill])
    dst_f = jnp.concatenate([dst, fill])
    bsrc, bdst = _bin_call(src_f, dst_f)
    src_p = bsrc.reshape(TOTBLK, BLK)
    dst_p = bdst.reshape(TOTBLK, BLK)
    x_p = jnp.pad(x, ((0, NPAD - N), (0, 0)))
    zeros1 = jnp.zeros((NPAD,), jnp.float32)
    zeros2 = jnp.zeros((NPAD, D), jnp.float32)

    hist = _deg_call(dst_p, zeros1)
    g1, dinv = _m1_call(x_p, W1, hist.T)
    s1 = _msg_call(g1, src_p, dst_p, zeros2)
    g2 = _m2_call(s1, g1, dinv, b1.reshape(1, D), W2)
    s2 = _msg_call(g2, src_p, dst_p, zeros2)
    out = _m3_call(s2, g2, dinv, b2.reshape(1, D), Wp, bp.reshape(1, D))
    return out[:N]


# speedup vs baseline: 1.0619x; 1.0046x over previous
import functools
import jax
import jax.numpy as jnp
from jax import lax
from jax.experimental import pallas as pl
from jax.experimental.pallas import tpu as pltpu
from jax.experimental.pallas import tpu_sc as plsc

N = 10000
NPAD = 10240
D = 128
NC = 2
NS = 16
NW = NC * NS
BLK = 128
NBLK = 80
EPAD = NW * NBLK * BLK
RPT = NPAD // NS
SENT = 10100
RB = 512
GRID = NPAD // RB

_mesh = plsc.VectorSubcoreMesh(core_axis_name="c", subcore_axis_name="s")

SH = 6
NBINS = 10240 >> SH
E0 = 128 * BLK
E1 = 32 * BLK


def _bin_tile(ecnt, ebase, srcf, dstf, outs, outd,
              src_in, dst_in, src_out, dst_out, hist, basep):
    pltpu.sync_copy(srcf.at[pl.ds(ebase, ecnt)], src_in.at[pl.ds(0, ecnt)])
    pltpu.sync_copy(dstf.at[pl.ds(ebase, ecnt)], dst_in.at[pl.ds(0, ecnt)])
    lanes = lax.iota(jnp.int32, 16)
    zeros16 = jnp.zeros((16,), jnp.int32)
    ones16 = jnp.ones((16,), jnp.int32)

    @pl.loop(0, NBINS)
    def _(cb):
        plsc.store_scatter(hist, [lanes, jnp.full((16,), cb, jnp.int32)],
                           zeros16)

    @pl.loop(0, ecnt // 16)
    def _(v):
        sv = src_in[pl.ds(v * 16, 16)]
        b = lax.shift_right_logical(sv, SH)
        plsc.addupdate_scatter(hist, [lanes, b], ones16)

    def pbody(cb, carry):
        bvec = jnp.full((16,), cb, jnp.int32)
        col = plsc.load_gather(hist, [lanes, bvec])
        inc = plsc.cumsum(col)
        plsc.store_scatter(basep, [lanes, bvec], carry + inc - col)
        return carry + jnp.sum(col)

    lax.fori_loop(0, NBINS, pbody, jnp.int32(0))

    @pl.loop(0, ecnt // 16)
    def _(v):
        sv = src_in[pl.ds(v * 16, 16)]
        dv = dst_in[pl.ds(v * 16, 16)]
        b = lax.shift_right_logical(sv, SH)
        pos = plsc.load_gather(basep, [lanes, b])
        plsc.store_scatter(src_out, [pos], sv)
        plsc.store_scatter(dst_out, [pos], dv)
        plsc.store_scatter(basep, [lanes, b], pos + ones16)

    pltpu.sync_copy(src_out.at[pl.ds(0, ecnt)], outs.at[pl.ds(ebase, ecnt)])
    pltpu.sync_copy(dst_out.at[pl.ds(0, ecnt)], outd.at[pl.ds(ebase, ecnt)])


def _bin_body(srcf, dstf, outs, outd,
              src_in, dst_in, src_out, dst_out, hist, basep):
    c = lax.axis_index("c")
    s = lax.axis_index("s")

    @pl.when(c == 0)
    def _():
        _bin_tile(E0, s * E0, srcf, dstf, outs, outd,
                  src_in, dst_in, src_out, dst_out, hist, basep)

    @pl.when(c == 1)
    def _():
        _bin_tile(E1, NS * E0 + s * E1, srcf, dstf, outs, outd,
                  src_in, dst_in, src_out, dst_out, hist, basep)


_bin_call = pl.kernel(
    _bin_body,
    compiler_params=pltpu.CompilerParams(needs_layout_passes=False),
    out_type=[
        jax.ShapeDtypeStruct((NW * NBLK * BLK,), jnp.int32),
        jax.ShapeDtypeStruct((NW * NBLK * BLK,), jnp.int32),
    ],
    mesh=_mesh,
    scratch_types=[
        pltpu.VMEM((E0,), jnp.int32),
        pltpu.VMEM((E0,), jnp.int32),
        pltpu.VMEM((E0,), jnp.int32),
        pltpu.VMEM((E0,), jnp.int32),
        pltpu.VMEM((16, NBINS), jnp.int32),
        pltpu.VMEM((16, NBINS), jnp.int32),
    ],
)


def _deg_body(dstb_hbm, zeros1_hbm, out_hbm, dst_v, ones_v, hist_sh, sem):
    c = lax.axis_index("c")
    s = lax.axis_index("s")
    wid = c * NS + s
    pltpu.sync_copy(dstb_hbm.at[pl.ds(wid * NBLK, NBLK)], dst_v)
    for k in range(BLK // 16):
        ones_v[pl.ds(k * 16, 16)] = jnp.ones((16,), jnp.float32)

    @pl.when(s == 0)
    def _():
        pltpu.sync_copy(zeros1_hbm, hist_sh)

    plsc.subcore_barrier()

    def body(j, carry):
        pltpu.async_copy(ones_v, hist_sh.at[dst_v.at[j]], sem, add=True).wait()
        return carry

    lax.fori_loop(0, NBLK, body, 0)
    plsc.subcore_barrier()

    @pl.when(s == 0)
    def _():
        pltpu.sync_copy(hist_sh, out_hbm.at[c])


_deg_call = pl.kernel(
    _deg_body,
    out_type=jax.ShapeDtypeStruct((NC, NPAD), jnp.float32),
    mesh=_mesh,
    scratch_types=[
        pltpu.VMEM((NBLK, BLK), jnp.int32),
        pltpu.VMEM((BLK,), jnp.float32),
        pltpu.VMEM_SHARED((NPAD,), jnp.float32),
        pltpu.SemaphoreType.DMA,
    ],
)


NBUF = 2
CHUNK = 16
B0 = 128
B1 = 32
TOTBLK = NS * (B0 + B1)


def _msg_ring(nb, base, g_hbm, srcb_hbm, dstb_hbm,
              src_v, dst_v, rows_v, acc_sh, gsems, ssems):
    def start_gather(j, b):
        pltpu.async_copy(g_hbm.at[src_v.at[j]], rows_v.at[b], gsems[b])

    def wait_gather(j, b):
        pltpu.make_async_copy(g_hbm.at[src_v.at[j]], rows_v.at[b],
                              gsems[b]).wait()

    def start_scatter(j, b):
        pltpu.async_copy(rows_v.at[b], acc_sh.at[dst_v.at[j]], ssems[b],
                         add=True)

    def wait_scatter(j, b):
        pltpu.make_async_copy(rows_v.at[b], acc_sh.at[dst_v.at[j]],
                              ssems[b]).wait()

    for h in range(nb // CHUNK):
        pltpu.sync_copy(srcb_hbm.at[pl.ds(base + h * CHUNK, CHUNK)], src_v)
        pltpu.sync_copy(dstb_hbm.at[pl.ds(base + h * CHUNK, CHUNK)], dst_v)

        for b in range(NBUF - 1):
            start_gather(b, b)
        for j in range(NBUF):
            wait_gather(j, j)
            start_scatter(j, j)
            if j > 0:
                wait_scatter(j - 1, (j - 1) % NBUF)
            start_gather(j + NBUF - 1, (j + NBUF - 1) % NBUF)

        @pl.loop(NBUF, CHUNK, step=NBUF)
        def _(j0):
            for b in range(NBUF):
                j = j0 + b
                wait_gather(j, b)
                start_scatter(j, b)
                nb2 = (b + NBUF - 1) % NBUF

                @pl.when(j + NBUF - 1 < CHUNK)
                def _():
                    wait_scatter(j - 1, nb2)
                    start_gather(j + NBUF - 1, nb2)

        for b in range(NBUF):
            j = CHUNK - NBUF + b
            wait_scatter(j, j % NBUF)


def _msg_body(g_hbm, srcb_hbm, dstb_hbm, zeros2_hbm, out_hbm,
              src_v, dst_v, rows_v, acc_sh, *sems):
    gsems = sems[:NBUF]
    ssems = sems[NBUF:]
    c = lax.axis_index("c")
    s = lax.axis_index("s")
    pltpu.sync_copy(zeros2_hbm.at[pl.ds(s * RPT, RPT)],
                    acc_sh.at[pl.ds(s * RPT, RPT)])
    plsc.subcore_barrier()

    @pl.when(c == 0)
    def _():
        _msg_ring(B0, s * B0, g_hbm, srcb_hbm, dstb_hbm,
                  src_v, dst_v, rows_v, acc_sh, gsems, ssems)

    @pl.when(c == 1)
    def _():
        _msg_ring(B1, NS * B0 + s * B1, g_hbm, srcb_hbm, dstb_hbm,
                  src_v, dst_v, rows_v, acc_sh, gsems, ssems)

    plsc.subcore_barrier()
    pltpu.sync_copy(acc_sh.at[pl.ds(s * RPT, RPT)],
                    out_hbm.at[c, pl.ds(s * RPT, RPT)])


_msg_call = pl.kernel(
    _msg_body,
    out_type=jax.ShapeDtypeStruct((NC, NPAD, D), jnp.float32),
    mesh=_mesh,
    scratch_types=[
        pltpu.VMEM((CHUNK, BLK), jnp.int32),
        pltpu.VMEM((CHUNK, BLK), jnp.int32),
        pltpu.VMEM((NBUF, BLK, D), jnp.float32),
        pltpu.VMEM_SHARED((NPAD, D), jnp.float32),
    ] + [pltpu.SemaphoreType.DMA] * (2 * NBUF),
)


def _m1_body(x_ref, w_ref, hist_ref, g_ref, dinv_ref):
    i = pl.program_id(0)
    h = jnp.dot(x_ref[...], w_ref[...], preferred_element_type=jnp.float32)
    deg = hist_ref[...].sum(axis=1, keepdims=True) + 1.0
    rid = lax.broadcasted_iota(jnp.int32, (RB, 1), 0) + i * RB
    dinv = jnp.where(rid < N, lax.rsqrt(deg), 0.0)
    g_ref[...] = dinv * h
    dinv_ref[...] = dinv


_m1_call = pl.pallas_call(
    _m1_body,
    grid=(GRID,),
    in_specs=[
        pl.BlockSpec((RB, D), lambda i: (i, 0)),
        pl.BlockSpec((D, D), lambda i: (0, 0)),
        pl.BlockSpec((RB, NC), lambda i: (i, 0)),
    ],
    out_specs=[
        pl.BlockSpec((RB, D), lambda i: (i, 0)),
        pl.BlockSpec((RB, 1), lambda i: (i, 0)),
    ],
    out_shape=[
        jax.ShapeDtypeStruct((NPAD, D), jnp.float32),
        jax.ShapeDtypeStruct((NPAD, 1), jnp.float32),
    ],
)


def _m2_body(s_ref, g_ref, dinv_ref, b_ref, w_ref, out_ref):
    dinv = dinv_ref[...]
    z = jnp.maximum(dinv * (s_ref[0] + s_ref[1] + g_ref[...]) + b_ref[...],
                    0.0)
    out_ref[...] = dinv * jnp.dot(z, w_ref[...],
                                  preferred_element_type=jnp.float32)


_m2_call = pl.pallas_call(
    _m2_body,
    grid=(GRID,),
    in_specs=[
        pl.BlockSpec((NC, RB, D), lambda i: (0, i, 0)),
        pl.BlockSpec((RB, D), lambda i: (i, 0)),
        pl.BlockSpec((RB, 1), lambda i: (i, 0)),
        pl.BlockSpec((1, D), lambda i: (0, 0)),
        pl.BlockSpec((D, D), lambda i: (0, 0)),
    ],
    out_specs=pl.BlockSpec((RB, D), lambda i: (i, 0)),
    out_shape=jax.ShapeDtypeStruct((NPAD, D), jnp.float32),
)


def _m3_body(s_ref, g_ref, dinv_ref, b_ref, w_ref, bp_ref, out_ref):
    h = (dinv_ref[...] * (s_ref[0] + s_ref[1] + g_ref[...]) + b_ref[...])
    out_ref[...] = jnp.dot(h, w_ref[...],
                           preferred_element_type=jnp.float32) + bp_ref[...]


_m3_call = pl.pallas_call(
    _m3_body,
    grid=(GRID,),
    in_specs=[
        pl.BlockSpec((NC, RB, D), lambda i: (0, i, 0)),
        pl.BlockSpec((RB, D), lambda i: (i, 0)),
        pl.BlockSpec((RB, 1), lambda i: (i, 0)),
        pl.BlockSpec((1, D), lambda i: (0, 0)),
        pl.BlockSpec((D, D), lambda i: (0, 0)),
        pl.BlockSpec((1, D), lambda i: (0, 0)),
    ],
    out_specs=pl.BlockSpec((RB, D), lambda i: (i, 0)),
    out_shape=jax.ShapeDtypeStruct((NPAD, D), jnp.float32),
)


@jax.jit
def kernel(x, edge_index, W1, b1, W2, b2, Wp, bp):
    src = edge_index[0].astype(jnp.int32)
    dst = edge_index[1].astype(jnp.int32)
    e = src.shape[0]
    fill = jnp.full((EPAD - e,), SENT, jnp.int32)
    src_f = jnp.concatenate([src, fill])
    dst_f = jnp.concatenate([dst, fill])
    bsrc, bdst = _bin_call(src_f, dst_f)
    src_p = bsrc.reshape(TOTBLK, BLK)
    dst_p = bdst.reshape(TOTBLK, BLK)
    x_p = jnp.pad(x, ((0, NPAD - N), (0, 0)))
    zeros1 = jnp.zeros((NPAD,), jnp.float32)
    zeros2 = jnp.zeros((NPAD, D), jnp.float32)

    hist = _deg_call(dst_p, zeros1)
    g1, dinv = _m1_call(x_p, W1, hist.T)
    s1 = _msg_call(g1, src_p, dst_p, zeros2)
    g2 = _m2_call(s1, g1, dinv, b1.reshape(1, D), W2)
    s2 = _msg_call(g2, src_p, dst_p, zeros2)
    out = _m3_call(s2, g2, dinv, b2.reshape(1, D), Wp, bp.reshape(1, D))
    return out[:N]


# speedup vs baseline: 1.0676x; 1.0054x over previous
import functools
import jax
import jax.numpy as jnp
from jax import lax
from jax.experimental import pallas as pl
from jax.experimental.pallas import tpu as pltpu
from jax.experimental.pallas import tpu_sc as plsc

N = 10000
NPAD = 10240
D = 128
NC = 2
NS = 16
NW = NC * NS
BLK = 128
NBLK = 80
EPAD = NW * NBLK * BLK
RPT = NPAD // NS
SENT = 10100
RB = 512
GRID = NPAD // RB

_mesh = plsc.VectorSubcoreMesh(core_axis_name="c", subcore_axis_name="s")

SH = 7
NBINS = 10240 >> SH
E0 = 128 * BLK
E1 = 32 * BLK


def _bin_tile(ecnt, ebase, srcf, dstf, outs, outd,
              src_in, dst_in, src_out, dst_out, hist, basep):
    pltpu.sync_copy(srcf.at[pl.ds(ebase, ecnt)], src_in.at[pl.ds(0, ecnt)])
    pltpu.sync_copy(dstf.at[pl.ds(ebase, ecnt)], dst_in.at[pl.ds(0, ecnt)])
    lanes = lax.iota(jnp.int32, 16)
    zeros16 = jnp.zeros((16,), jnp.int32)
    ones16 = jnp.ones((16,), jnp.int32)

    @pl.loop(0, NBINS)
    def _(cb):
        plsc.store_scatter(hist, [lanes, jnp.full((16,), cb, jnp.int32)],
                           zeros16)

    @pl.loop(0, ecnt // 16)
    def _(v):
        sv = src_in[pl.ds(v * 16, 16)]
        b = lax.shift_right_logical(sv, SH)
        plsc.addupdate_scatter(hist, [lanes, b], ones16)

    def pbody(cb, carry):
        bvec = jnp.full((16,), cb, jnp.int32)
        col = plsc.load_gather(hist, [lanes, bvec])
        inc = plsc.cumsum(col)
        plsc.store_scatter(basep, [lanes, bvec], carry + inc - col)
        return carry + jnp.sum(col)

    lax.fori_loop(0, NBINS, pbody, jnp.int32(0))

    @pl.loop(0, ecnt // 16)
    def _(v):
        sv = src_in[pl.ds(v * 16, 16)]
        dv = dst_in[pl.ds(v * 16, 16)]
        b = lax.shift_right_logical(sv, SH)
        pos = plsc.load_gather(basep, [lanes, b])
        plsc.store_scatter(src_out, [pos], sv)
        plsc.store_scatter(dst_out, [pos], dv)
        plsc.store_scatter(basep, [lanes, b], pos + ones16)

    pltpu.sync_copy(src_out.at[pl.ds(0, ecnt)], outs.at[pl.ds(ebase, ecnt)])
    pltpu.sync_copy(dst_out.at[pl.ds(0, ecnt)], outd.at[pl.ds(ebase, ecnt)])


def _bin_body(srcf, dstf, outs, outd,
              src_in, dst_in, src_out, dst_out, hist, basep):
    c = lax.axis_index("c")
    s = lax.axis_index("s")

    @pl.when(c == 0)
    def _():
        _bin_tile(E0, s * E0, srcf, dstf, outs, outd,
                  src_in, dst_in, src_out, dst_out, hist, basep)

    @pl.when(c == 1)
    def _():
        _bin_tile(E1, NS * E0 + s * E1, srcf, dstf, outs, outd,
                  src_in, dst_in, src_out, dst_out, hist, basep)


_bin_call = pl.kernel(
    _bin_body,
    compiler_params=pltpu.CompilerParams(needs_layout_passes=False),
    out_type=[
        jax.ShapeDtypeStruct((NW * NBLK * BLK,), jnp.int32),
        jax.ShapeDtypeStruct((NW * NBLK * BLK,), jnp.int32),
    ],
    mesh=_mesh,
    scratch_types=[
        pltpu.VMEM((E0,), jnp.int32),
        pltpu.VMEM((E0,), jnp.int32),
        pltpu.VMEM((E0,), jnp.int32),
        pltpu.VMEM((E0,), jnp.int32),
        pltpu.VMEM((16, NBINS), jnp.int32),
        pltpu.VMEM((16, NBINS), jnp.int32),
    ],
)


def _deg_body(dstb_hbm, zeros1_hbm, out_hbm, dst_v, ones_v, hist_sh, sem):
    c = lax.axis_index("c")
    s = lax.axis_index("s")
    wid = c * NS + s
    pltpu.sync_copy(dstb_hbm.at[pl.ds(wid * NBLK, NBLK)], dst_v)
    for k in range(BLK // 16):
        ones_v[pl.ds(k * 16, 16)] = jnp.ones((16,), jnp.float32)

    @pl.when(s == 0)
    def _():
        pltpu.sync_copy(zeros1_hbm, hist_sh)

    plsc.subcore_barrier()

    def body(j, carry):
        pltpu.async_copy(ones_v, hist_sh.at[dst_v.at[j]], sem, add=True).wait()
        return carry

    lax.fori_loop(0, NBLK, body, 0)
    plsc.subcore_barrier()

    @pl.when(s == 0)
    def _():
        pltpu.sync_copy(hist_sh, out_hbm.at[c])


_deg_call = pl.kernel(
    _deg_body,
    out_type=jax.ShapeDtypeStruct((NC, NPAD), jnp.float32),
    mesh=_mesh,
    scratch_types=[
        pltpu.VMEM((NBLK, BLK), jnp.int32),
        pltpu.VMEM((BLK,), jnp.float32),
        pltpu.VMEM_SHARED((NPAD,), jnp.float32),
        pltpu.SemaphoreType.DMA,
    ],
)


NBUF = 2
CHUNK = 16
B0 = 128
B1 = 32
TOTBLK = NS * (B0 + B1)


def _msg_ring(nb, base, g_hbm, srcb_hbm, dstb_hbm,
              src_v, dst_v, rows_v, acc_sh, gsems, ssems):
    def start_gather(j, b):
        pltpu.async_copy(g_hbm.at[src_v.at[j]], rows_v.at[b], gsems[b])

    def wait_gather(j, b):
        pltpu.make_async_copy(g_hbm.at[src_v.at[j]], rows_v.at[b],
                              gsems[b]).wait()

    def start_scatter(j, b):
        pltpu.async_copy(rows_v.at[b], acc_sh.at[dst_v.at[j]], ssems[b],
                         add=True)

    def wait_scatter(j, b):
        pltpu.make_async_copy(rows_v.at[b], acc_sh.at[dst_v.at[j]],
                              ssems[b]).wait()

    for h in range(nb // CHUNK):
        pltpu.sync_copy(srcb_hbm.at[pl.ds(base + h * CHUNK, CHUNK)], src_v)
        pltpu.sync_copy(dstb_hbm.at[pl.ds(base + h * CHUNK, CHUNK)], dst_v)

        for b in range(NBUF - 1):
            start_gather(b, b)
        for j in range(NBUF):
            wait_gather(j, j)
            start_scatter(j, j)
            if j > 0:
                wait_scatter(j - 1, (j - 1) % NBUF)
            start_gather(j + NBUF - 1, (j + NBUF - 1) % NBUF)

        @pl.loop(NBUF, CHUNK, step=NBUF)
        def _(j0):
            for b in range(NBUF):
                j = j0 + b
                wait_gather(j, b)
                start_scatter(j, b)
                nb2 = (b + NBUF - 1) % NBUF

                @pl.when(j + NBUF - 1 < CHUNK)
                def _():
                    wait_scatter(j - 1, nb2)
                    start_gather(j + NBUF - 1, nb2)

        for b in range(NBUF):
            j = CHUNK - NBUF + b
            wait_scatter(j, j % NBUF)


def _msg_body(g_hbm, srcb_hbm, dstb_hbm, zeros2_hbm, out_hbm,
              src_v, dst_v, rows_v, acc_sh, *sems):
    gsems = sems[:NBUF]
    ssems = sems[NBUF:]
    c = lax.axis_index("c")
    s = lax.axis_index("s")
    pltpu.sync_copy(zeros2_hbm.at[pl.ds(s * RPT, RPT)],
                    acc_sh.at[pl.ds(s * RPT, RPT)])
    plsc.subcore_barrier()

    @pl.when(c == 0)
    def _():
        _msg_ring(B0, s * B0, g_hbm, srcb_hbm, dstb_hbm,
                  src_v, dst_v, rows_v, acc_sh, gsems, ssems)

    @pl.when(c == 1)
    def _():
        _msg_ring(B1, NS * B0 + s * B1, g_hbm, srcb_hbm, dstb_hbm,
                  src_v, dst_v, rows_v, acc_sh, gsems, ssems)

    plsc.subcore_barrier()
    pltpu.sync_copy(acc_sh.at[pl.ds(s * RPT, RPT)],
                    out_hbm.at[c, pl.ds(s * RPT, RPT)])


_msg_call = pl.kernel(
    _msg_body,
    out_type=jax.ShapeDtypeStruct((NC, NPAD, D), jnp.float32),
    mesh=_mesh,
    scratch_types=[
        pltpu.VMEM((CHUNK, BLK), jnp.int32),
        pltpu.VMEM((CHUNK, BLK), jnp.int32),
        pltpu.VMEM((NBUF, BLK, D), jnp.float32),
        pltpu.VMEM_SHARED((NPAD, D), jnp.float32),
    ] + [pltpu.SemaphoreType.DMA] * (2 * NBUF),
)


def _m1_body(x_ref, w_ref, hist_ref, g_ref, dinv_ref):
    i = pl.program_id(0)
    h = jnp.dot(x_ref[...], w_ref[...], preferred_element_type=jnp.float32)
    deg = hist_ref[...].sum(axis=1, keepdims=True) + 1.0
    rid = lax.broadcasted_iota(jnp.int32, (RB, 1), 0) + i * RB
    dinv = jnp.where(rid < N, lax.rsqrt(deg), 0.0)
    g_ref[...] = dinv * h
    dinv_ref[...] = dinv


_m1_call = pl.pallas_call(
    _m1_body,
    grid=(GRID,),
    in_specs=[
        pl.BlockSpec((RB, D), lambda i: (i, 0)),
        pl.BlockSpec((D, D), lambda i: (0, 0)),
        pl.BlockSpec((RB, NC), lambda i: (i, 0)),
    ],
    out_specs=[
        pl.BlockSpec((RB, D), lambda i: (i, 0)),
        pl.BlockSpec((RB, 1), lambda i: (i, 0)),
    ],
    out_shape=[
        jax.ShapeDtypeStruct((NPAD, D), jnp.float32),
        jax.ShapeDtypeStruct((NPAD, 1), jnp.float32),
    ],
)


def _m2_body(s_ref, g_ref, dinv_ref, b_ref, w_ref, out_ref):
    dinv = dinv_ref[...]
    z = jnp.maximum(dinv * (s_ref[0] + s_ref[1] + g_ref[...]) + b_ref[...],
                    0.0)
    out_ref[...] = dinv * jnp.dot(z, w_ref[...],
                                  preferred_element_type=jnp.float32)


_m2_call = pl.pallas_call(
    _m2_body,
    grid=(GRID,),
    in_specs=[
        pl.BlockSpec((NC, RB, D), lambda i: (0, i, 0)),
        pl.BlockSpec((RB, D), lambda i: (i, 0)),
        pl.BlockSpec((RB, 1), lambda i: (i, 0)),
        pl.BlockSpec((1, D), lambda i: (0, 0)),
        pl.BlockSpec((D, D), lambda i: (0, 0)),
    ],
    out_specs=pl.BlockSpec((RB, D), lambda i: (i, 0)),
    out_shape=jax.ShapeDtypeStruct((NPAD, D), jnp.float32),
)


def _m3_body(s_ref, g_ref, dinv_ref, b_ref, w_ref, bp_ref, out_ref):
    h = (dinv_ref[...] * (s_ref[0] + s_ref[1] + g_ref[...]) + b_ref[...])
    out_ref[...] = jnp.dot(h, w_ref[...],
                           preferred_element_type=jnp.float32) + bp_ref[...]


_m3_call = pl.pallas_call(
    _m3_body,
    grid=(GRID,),
    in_specs=[
        pl.BlockSpec((NC, RB, D), lambda i: (0, i, 0)),
        pl.BlockSpec((RB, D), lambda i: (i, 0)),
        pl.BlockSpec((RB, 1), lambda i: (i, 0)),
        pl.BlockSpec((1, D), lambda i: (0, 0)),
        pl.BlockSpec((D, D), lambda i: (0, 0)),
        pl.BlockSpec((1, D), lambda i: (0, 0)),
    ],
    out_specs=pl.BlockSpec((RB, D), lambda i: (i, 0)),
    out_shape=jax.ShapeDtypeStruct((NPAD, D), jnp.float32),
)


@jax.jit
def kernel(x, edge_index, W1, b1, W2, b2, Wp, bp):
    src = edge_index[0].astype(jnp.int32)
    dst = edge_index[1].astype(jnp.int32)
    e = src.shape[0]
    fill = jnp.full((EPAD - e,), SENT, jnp.int32)
    src_f = jnp.concatenate([src, fill])
    dst_f = jnp.concatenate([dst, fill])
    bsrc, bdst = _bin_call(src_f, dst_f)
    src_p = bsrc.reshape(TOTBLK, BLK)
    dst_p = bdst.reshape(TOTBLK, BLK)
    x_p = jnp.pad(x, ((0, NPAD - N), (0, 0)))
    zeros1 = jnp.zeros((NPAD,), jnp.float32)
    zeros2 = jnp.zeros((NPAD, D), jnp.float32)

    hist = _deg_call(dst_p, zeros1)
    g1, dinv = _m1_call(x_p, W1, hist.T)
    s1 = _msg_call(g1, src_p, dst_p, zeros2)
    g2 = _m2_call(s1, g1, dinv, b1.reshape(1, D), W2)
    s2 = _msg_call(g2, src_p, dst_p, zeros2)
    out = _m3_call(s2, g2, dinv, b2.reshape(1, D), Wp, bp.reshape(1, D))
    return out[:N]


# speedup vs baseline: 1.0681x; 1.0005x over previous
import functools
import jax
import jax.numpy as jnp
from jax import lax
from jax.experimental import pallas as pl
from jax.experimental.pallas import tpu as pltpu
from jax.experimental.pallas import tpu_sc as plsc

N = 10000
NPAD = 10240
D = 128
NC = 2
NS = 16
NW = NC * NS
BLK = 128
NBLK = 80
EPAD = NW * NBLK * BLK
RPT = NPAD // NS
SENT = 10100
RB = 512
GRID = NPAD // RB

_mesh = plsc.VectorSubcoreMesh(core_axis_name="c", subcore_axis_name="s")

SH = 8
NBINS = 10240 >> SH
E0 = 128 * BLK
E1 = 32 * BLK


def _bin_tile(ecnt, ebase, srcf, dstf, outs, outd,
              src_in, dst_in, src_out, dst_out, hist, basep):
    pltpu.sync_copy(srcf.at[pl.ds(ebase, ecnt)], src_in.at[pl.ds(0, ecnt)])
    pltpu.sync_copy(dstf.at[pl.ds(ebase, ecnt)], dst_in.at[pl.ds(0, ecnt)])
    lanes = lax.iota(jnp.int32, 16)
    zeros16 = jnp.zeros((16,), jnp.int32)
    ones16 = jnp.ones((16,), jnp.int32)

    @pl.loop(0, NBINS)
    def _(cb):
        plsc.store_scatter(hist, [lanes, jnp.full((16,), cb, jnp.int32)],
                           zeros16)

    @pl.loop(0, ecnt // 16)
    def _(v):
        sv = src_in[pl.ds(v * 16, 16)]
        b = lax.shift_right_logical(sv, SH)
        plsc.addupdate_scatter(hist, [lanes, b], ones16)

    def pbody(cb, carry):
        bvec = jnp.full((16,), cb, jnp.int32)
        col = plsc.load_gather(hist, [lanes, bvec])
        inc = plsc.cumsum(col)
        plsc.store_scatter(basep, [lanes, bvec], carry + inc - col)
        return carry + jnp.sum(col)

    lax.fori_loop(0, NBINS, pbody, jnp.int32(0))

    @pl.loop(0, ecnt // 16)
    def _(v):
        sv = src_in[pl.ds(v * 16, 16)]
        dv = dst_in[pl.ds(v * 16, 16)]
        b = lax.shift_right_logical(sv, SH)
        pos = plsc.load_gather(basep, [lanes, b])
        plsc.store_scatter(src_out, [pos], sv)
        plsc.store_scatter(dst_out, [pos], dv)
        plsc.store_scatter(basep, [lanes, b], pos + ones16)

    pltpu.sync_copy(src_out.at[pl.ds(0, ecnt)], outs.at[pl.ds(ebase, ecnt)])
    pltpu.sync_copy(dst_out.at[pl.ds(0, ecnt)], outd.at[pl.ds(ebase, ecnt)])


def _bin_body(srcf, dstf, outs, outd,
              src_in, dst_in, src_out, dst_out, hist, basep):
    c = lax.axis_index("c")
    s = lax.axis_index("s")

    @pl.when(c == 0)
    def _():
        _bin_tile(E0, s * E0, srcf, dstf, outs, outd,
                  src_in, dst_in, src_out, dst_out, hist, basep)

    @pl.when(c == 1)
    def _():
        _bin_tile(E1, NS * E0 + s * E1, srcf, dstf, outs, outd,
                  src_in, dst_in, src_out, dst_out, hist, basep)


_bin_call = pl.kernel(
    _bin_body,
    compiler_params=pltpu.CompilerParams(needs_layout_passes=False),
    out_type=[
        jax.ShapeDtypeStruct((NW * NBLK * BLK,), jnp.int32),
        jax.ShapeDtypeStruct((NW * NBLK * BLK,), jnp.int32),
    ],
    mesh=_mesh,
    scratch_types=[
        pltpu.VMEM((E0,), jnp.int32),
        pltpu.VMEM((E0,), jnp.int32),
        pltpu.VMEM((E0,), jnp.int32),
        pltpu.VMEM((E0,), jnp.int32),
        pltpu.VMEM((16, NBINS), jnp.int32),
        pltpu.VMEM((16, NBINS), jnp.int32),
    ],
)


def _deg_body(dstb_hbm, zeros1_hbm, out_hbm, dst_v, ones_v, hist_sh, sem):
    c = lax.axis_index("c")
    s = lax.axis_index("s")
    wid = c * NS + s
    pltpu.sync_copy(dstb_hbm.at[pl.ds(wid * NBLK, NBLK)], dst_v)
    for k in range(BLK // 16):
        ones_v[pl.ds(k * 16, 16)] = jnp.ones((16,), jnp.float32)

    @pl.when(s == 0)
    def _():
        pltpu.sync_copy(zeros1_hbm, hist_sh)

    plsc.subcore_barrier()

    def body(j, carry):
        pltpu.async_copy(ones_v, hist_sh.at[dst_v.at[j]], sem, add=True).wait()
        return carry

    lax.fori_loop(0, NBLK, body, 0)
    plsc.subcore_barrier()

    @pl.when(s == 0)
    def _():
        pltpu.sync_copy(hist_sh, out_hbm.at[c])


_deg_call = pl.kernel(
    _deg_body,
    out_type=jax.ShapeDtypeStruct((NC, NPAD), jnp.float32),
    mesh=_mesh,
    scratch_types=[
        pltpu.VMEM((NBLK, BLK), jnp.int32),
        pltpu.VMEM((BLK,), jnp.float32),
        pltpu.VMEM_SHARED((NPAD,), jnp.float32),
        pltpu.SemaphoreType.DMA,
    ],
)


NBUF = 2
CHUNK = 16
B0 = 128
B1 = 32
TOTBLK = NS * (B0 + B1)


def _msg_ring(nb, base, g_hbm, srcb_hbm, dstb_hbm,
              src_v, dst_v, rows_v, acc_sh, gsems, ssems):
    def start_gather(j, b):
        pltpu.async_copy(g_hbm.at[src_v.at[j]], rows_v.at[b], gsems[b])

    def wait_gather(j, b):
        pltpu.make_async_copy(g_hbm.at[src_v.at[j]], rows_v.at[b],
                              gsems[b]).wait()

    def start_scatter(j, b):
        pltpu.async_copy(rows_v.at[b], acc_sh.at[dst_v.at[j]], ssems[b],
                         add=True)

    def wait_scatter(j, b):
        pltpu.make_async_copy(rows_v.at[b], acc_sh.at[dst_v.at[j]],
                              ssems[b]).wait()

    for h in range(nb // CHUNK):
        pltpu.sync_copy(srcb_hbm.at[pl.ds(base + h * CHUNK, CHUNK)], src_v)
        pltpu.sync_copy(dstb_hbm.at[pl.ds(base + h * CHUNK, CHUNK)], dst_v)

        for b in range(NBUF - 1):
            start_gather(b, b)
        for j in range(NBUF):
            wait_gather(j, j)
            start_scatter(j, j)
            if j > 0:
                wait_scatter(j - 1, (j - 1) % NBUF)
            start_gather(j + NBUF - 1, (j + NBUF - 1) % NBUF)

        @pl.loop(NBUF, CHUNK, step=NBUF)
        def _(j0):
            for b in range(NBUF):
                j = j0 + b
                wait_gather(j, b)
                start_scatter(j, b)
                nb2 = (b + NBUF - 1) % NBUF

                @pl.when(j + NBUF - 1 < CHUNK)
                def _():
                    wait_scatter(j - 1, nb2)
                    start_gather(j + NBUF - 1, nb2)

        for b in range(NBUF):
            j = CHUNK - NBUF + b
            wait_scatter(j, j % NBUF)


def _msg_body(g_hbm, srcb_hbm, dstb_hbm, zeros2_hbm, out_hbm,
              src_v, dst_v, rows_v, acc_sh, *sems):
    gsems = sems[:NBUF]
    ssems = sems[NBUF:]
    c = lax.axis_index("c")
    s = lax.axis_index("s")
    pltpu.sync_copy(zeros2_hbm.at[pl.ds(s * RPT, RPT)],
                    acc_sh.at[pl.ds(s * RPT, RPT)])
    plsc.subcore_barrier()

    @pl.when(c == 0)
    def _():
        _msg_ring(B0, s * B0, g_hbm, srcb_hbm, dstb_hbm,
                  src_v, dst_v, rows_v, acc_sh, gsems, ssems)

    @pl.when(c == 1)
    def _():
        _msg_ring(B1, NS * B0 + s * B1, g_hbm, srcb_hbm, dstb_hbm,
                  src_v, dst_v, rows_v, acc_sh, gsems, ssems)

    plsc.subcore_barrier()
    pltpu.sync_copy(acc_sh.at[pl.ds(s * RPT, RPT)],
                    out_hbm.at[c, pl.ds(s * RPT, RPT)])


_msg_call = pl.kernel(
    _msg_body,
    out_type=jax.ShapeDtypeStruct((NC, NPAD, D), jnp.float32),
    mesh=_mesh,
    scratch_types=[
        pltpu.VMEM((CHUNK, BLK), jnp.int32),
        pltpu.VMEM((CHUNK, BLK), jnp.int32),
        pltpu.VMEM((NBUF, BLK, D), jnp.float32),
        pltpu.VMEM_SHARED((NPAD, D), jnp.float32),
    ] + [pltpu.SemaphoreType.DMA] * (2 * NBUF),
)


def _m1_body(x_ref, w_ref, hist_ref, g_ref, dinv_ref):
    i = pl.program_id(0)
    h = jnp.dot(x_ref[...], w_ref[...], preferred_element_type=jnp.float32)
    deg = hist_ref[...].sum(axis=1, keepdims=True) + 1.0
    rid = lax.broadcasted_iota(jnp.int32, (RB, 1), 0) + i * RB
    dinv = jnp.where(rid < N, lax.rsqrt(deg), 0.0)
    g_ref[...] = dinv * h
    dinv_ref[...] = dinv


_m1_call = pl.pallas_call(
    _m1_body,
    grid=(GRID,),
    in_specs=[
        pl.BlockSpec((RB, D), lambda i: (i, 0)),
        pl.BlockSpec((D, D), lambda i: (0, 0)),
        pl.BlockSpec((RB, NC), lambda i: (i, 0)),
    ],
    out_specs=[
        pl.BlockSpec((RB, D), lambda i: (i, 0)),
        pl.BlockSpec((RB, 1), lambda i: (i, 0)),
    ],
    out_shape=[
        jax.ShapeDtypeStruct((NPAD, D), jnp.float32),
        jax.ShapeDtypeStruct((NPAD, 1), jnp.float32),
    ],
)


def _m2_body(s_ref, g_ref, dinv_ref, b_ref, w_ref, out_ref):
    dinv = dinv_ref[...]
    z = jnp.maximum(dinv * (s_ref[0] + s_ref[1] + g_ref[...]) + b_ref[...],
                    0.0)
    out_ref[...] = dinv * jnp.dot(z, w_ref[...],
                                  preferred_element_type=jnp.float32)


_m2_call = pl.pallas_call(
    _m2_body,
    grid=(GRID,),
    in_specs=[
        pl.BlockSpec((NC, RB, D), lambda i: (0, i, 0)),
        pl.BlockSpec((RB, D), lambda i: (i, 0)),
        pl.BlockSpec((RB, 1), lambda i: (i, 0)),
        pl.BlockSpec((1, D), lambda i: (0, 0)),
        pl.BlockSpec((D, D), lambda i: (0, 0)),
    ],
    out_specs=pl.BlockSpec((RB, D), lambda i: (i, 0)),
    out_shape=jax.ShapeDtypeStruct((NPAD, D), jnp.float32),
)


def _m3_body(s_ref, g_ref, dinv_ref, b_ref, w_ref, bp_ref, out_ref):
    h = (dinv_ref[...] * (s_ref[0] + s_ref[1] + g_ref[...]) + b_ref[...])
    out_ref[...] = jnp.dot(h, w_ref[...],
                           preferred_element_type=jnp.float32) + bp_ref[...]


_m3_call = pl.pallas_call(
    _m3_body,
    grid=(GRID,),
    in_specs=[
        pl.BlockSpec((NC, RB, D), lambda i: (0, i, 0)),
        pl.BlockSpec((RB, D), lambda i: (i, 0)),
        pl.BlockSpec((RB, 1), lambda i: (i, 0)),
        pl.BlockSpec((1, D), lambda i: (0, 0)),
        pl.BlockSpec((D, D), lambda i: (0, 0)),
        pl.BlockSpec((1, D), lambda i: (0, 0)),
    ],
    out_specs=pl.BlockSpec((RB, D), lambda i: (i, 0)),
    out_shape=jax.ShapeDtypeStruct((NPAD, D), jnp.float32),
)


@jax.jit
def kernel(x, edge_index, W1, b1, W2, b2, Wp, bp):
    src = edge_index[0].astype(jnp.int32)
    dst = edge_index[1].astype(jnp.int32)
    e = src.shape[0]
    fill = jnp.full((EPAD - e,), SENT, jnp.int32)
    src_f = jnp.concatenate([src, fill])
    dst_f = jnp.concatenate([dst, fill])
    bsrc, bdst = _bin_call(src_f, dst_f)
    src_p = bsrc.reshape(TOTBLK, BLK)
    dst_p = bdst.reshape(TOTBLK, BLK)
    x_p = jnp.pad(x, ((0, NPAD - N), (0, 0)))
    zeros1 = jnp.zeros((NPAD,), jnp.float32)
    zeros2 = jnp.zeros((NPAD, D), jnp.float32)

    hist = _deg_call(dst_p, zeros1)
    g1, dinv = _m1_call(x_p, W1, hist.T)
    s1 = _msg_call(g1, src_p, dst_p, zeros2)
    g2 = _m2_call(s1, g1, dinv, b1.reshape(1, D), W2)
    s2 = _msg_call(g2, src_p, dst_p, zeros2)
    out = _m3_call(s2, g2, dinv, b2.reshape(1, D), Wp, bp.reshape(1, D))
    return out[:N]
